# baseline (speedup 1.0000x reference)
import jax
import jax.numpy as jnp
from jax import lax
from jax.experimental import pallas as pl
from jax.experimental.pallas import tpu as pltpu
from jax.experimental.pallas import tpu_sc as plsc

_N = 10000
_M = 10240
_D = 128
_NT = 32
_CHUNK = 64
_STG = 8
_RPT = _M // 16


def _tc1_body(x_ref, wz_ref, bz_ref, wh_ref, bh_ref, wg1_ref, hgru_ref, g1_ref):
    x = x_ref[...]
    z = jax.nn.sigmoid(
        jnp.dot(x, wz_ref[...], preferred_element_type=jnp.float32) + bz_ref[...])
    ht = jnp.tanh(
        jnp.dot(x, wh_ref[...], preferred_element_type=jnp.float32) + bh_ref[...])
    hg = z * ht
    hgru_ref[...] = hg
    g1_ref[...] = jnp.dot(hg, wg1_ref[...], preferred_element_type=jnp.float32)


def _inv_deg(degp_ref):
    deg = degp_ref[0, :, 0:1] + degp_ref[1, :, 0:1]
    return 1.0 / jnp.maximum(deg, 1.0)


def _tc2_body(p_ref, degp_ref, bg1_ref, wg2_ref, g2_ref):
    inv = _inv_deg(degp_ref)
    h1 = jax.nn.relu((p_ref[0, :, :] + p_ref[1, :, :]) * inv + bg1_ref[...])
    g2_ref[...] = jnp.dot(h1, wg2_ref[...], preferred_element_type=jnp.float32)


def _tc3_body(p_ref, degp_ref, bg2_ref, hgru_ref, wout_ref, bout_ref, out_ref):
    inv = _inv_deg(degp_ref)
    h2 = jax.nn.relu((p_ref[0, :, :] + p_ref[1, :, :]) * inv + bg2_ref[...])
    fused = 0.5 * (hgru_ref[...] + h2)
    out_ref[...] = (
        jnp.dot(fused, wout_ref[...], preferred_element_type=jnp.float32)
        + bout_ref[...])


def _tc1(x, wz, bz, wh, bh, wg1):
    return pl.pallas_call(
        _tc1_body,
        out_shape=(jax.ShapeDtypeStruct((_M, _D), jnp.float32),
                   jax.ShapeDtypeStruct((_M, _D), jnp.float32)),
    )(x, wz, bz, wh, bh, wg1)


def _tc2(parts, degp, bg1, wg2):
    return pl.pallas_call(
        _tc2_body,
        out_shape=jax.ShapeDtypeStruct((_M, _D), jnp.float32),
    )(parts, degp, bg1, wg2)


def _tc3(parts, degp, bg2, hgru, wout, bout):
    return pl.pallas_call(
        _tc3_body,
        out_shape=jax.ShapeDtypeStruct((_M, _D), jnp.float32),
    )(parts, degp, bg2, hgru, wout, bout)


def _sc_mesh():
    return plsc.VectorSubcoreMesh(core_axis_name="c", subcore_axis_name="s")


def _zero_fill(buf, rows):
    zv = jnp.zeros((16,), jnp.float32)

    def zrow(i, carry):
        for j in range(_D // 16):
            buf[i, pl.ds(j * 16, 16)] = zv
        return carry
    lax.fori_loop(0, rows, zrow, 0)


def _make_conv(cpt0: int, cpt1: int):
    assert cpt0 % _STG == 0 and cpt1 % _STG == 0

    scratch = [
        pltpu.VMEM((_STG, _CHUNK), jnp.int32),
        pltpu.VMEM((_STG, _CHUNK), jnp.int32),
        pltpu.VMEM((_CHUNK, _D), jnp.float32),
        pltpu.VMEM((_CHUNK, _D), jnp.float32),
        pltpu.VMEM((_CHUNK, _D), jnp.float32),
        pltpu.VMEM_SHARED((_M, _D), jnp.float32),
        pltpu.SemaphoreType.DMA,
        pltpu.SemaphoreType.DMA,
        pltpu.SemaphoreType.DMA,
        pltpu.SemaphoreType.DMA,
        pltpu.SemaphoreType.DMA,
        pltpu.SemaphoreType.DMA,
    ]

    def body(g_hbm, srcA, dstA, srcB, dstB, part_out, idx_s, idx_d,
             gbuf0, gbuf1, gbuf2, acc,
             gsem0, gsem1, gsem2, ssem0, ssem1, ssem2):
        cid = lax.axis_index("c")
        sid = lax.axis_index("s")
        bufs = (gbuf0, gbuf1, gbuf2)
        gsems = (gsem0, gsem1, gsem2)
        ssems = (ssem0, ssem1, ssem2)

        _zero_fill(gbuf0, _CHUNK)
        base = pl.multiple_of(sid * _RPT, _CHUNK)
        for k in range(_RPT // _CHUNK):
            pltpu.sync_copy(gbuf0, acc.at[pl.ds(base + k * _CHUNK, _CHUNK)])
        plsc.subcore_barrier()

        def run_edges(src_hbm, dst_hbm, nstg):
            def stage_body(s, carry):
                stg = pl.ds(s * _STG, _STG)
                pltpu.sync_copy(src_hbm.at[sid, stg], idx_s)
                pltpu.sync_copy(dst_hbm.at[sid, stg], idx_d)

                g = [None] * _STG
                sc = [None] * _STG
                g[0] = pltpu.async_copy(g_hbm.at[idx_s.at[0]], bufs[0], gsems[0])
                g[1] = pltpu.async_copy(g_hbm.at[idx_s.at[1]], bufs[1], gsems[1])
                for j in range(_STG):
                    p = j % 3
                    g[j].wait()
                    sc[j] = pltpu.async_copy(
                        bufs[p], acc.at[idx_d.at[j]], ssems[p], add=True)
                    if j + 2 < _STG:
                        q = (j + 2) % 3
                        if j >= 1:
                            sc[j - 1].wait()
                        g[j + 2] = pltpu.async_copy(
                            g_hbm.at[idx_s.at[j + 2]], bufs[q], gsems[q])
                sc[_STG - 2].wait()
                sc[_STG - 1].wait()
                return carry
            lax.fori_loop(0, nstg, stage_body, 0)

        @pl.when(cid == 0)
        def _():
            run_edges(srcA, dstA, cpt0 // _STG)

        @pl.when(cid == 1)
        def _():
            run_edges(srcB, dstB, cpt1 // _STG)

        plsc.subcore_barrier()

        for k in range(_RPT // _CHUNK):
            sl = pl.ds(base + k * _CHUNK, _CHUNK)
            pltpu.sync_copy(acc.at[sl], part_out.at[cid, sl])

    return pl.kernel(
        body,
        mesh=_sc_mesh(),
        out_type=jax.ShapeDtypeStruct((2, _M, _D), jnp.float32),
        scratch_types=scratch,
    )


def _make_deg(cpt: int):
    assert cpt % _STG == 0
    nstg = cpt // _STG

    scratch = [
        pltpu.VMEM((_STG, _CHUNK), jnp.int32),
        pltpu.VMEM((_CHUNK, _D), jnp.float32),
        pltpu.VMEM_SHARED((_M, _D), jnp.float32),
        pltpu.SemaphoreType.DMA,
        pltpu.SemaphoreType.DMA,
    ]

    def body(dst_hbm, deg_out, idx_d, obuf, dacc, ssem0, ssem1):
        cid = lax.axis_index("c")
        sid = lax.axis_index("s")
        wid = sid * 2 + cid
        ssems = (ssem0, ssem1)

        _zero_fill(obuf, _CHUNK)
        base = pl.multiple_of(sid * _RPT, _CHUNK)
        for k in range(_RPT // _CHUNK):
            pltpu.sync_copy(obuf, dacc.at[pl.ds(base + k * _CHUNK, _CHUNK)])

        ov = jnp.ones((16,), jnp.float32)

        def orow(i, carry):
            for j in range(_D // 16):
                obuf[i, pl.ds(j * 16, 16)] = ov
            return carry
        lax.fori_loop(0, _CHUNK, orow, 0)
        plsc.subcore_barrier()

        def stage_body(s, carry):
            pltpu.sync_copy(dst_hbm.at[wid, pl.ds(s * _STG, _STG)], idx_d)
            prev = None
            for j in range(_STG):
                cur = pltpu.async_copy(
                    obuf, dacc.at[idx_d.at[j]], ssems[j % 2], add=True)
                if prev is not None:
                    prev.wait()
                prev = cur
            prev.wait()
            return carry
        lax.fori_loop(0, nstg, stage_body, 0)
        plsc.subcore_barrier()

        for k in range(_RPT // _CHUNK):
            sl = pl.ds(base + k * _CHUNK, _CHUNK)
            pltpu.sync_copy(dacc.at[sl], deg_out.at[cid, sl])

    return pl.kernel(
        body,
        mesh=_sc_mesh(),
        out_type=jax.ShapeDtypeStruct((2, _M, _D), jnp.float32),
        scratch_types=scratch,
    )


def kernel(node_features, edge_index, W_r, b_r, W_z, b_z, W_h, b_h,
           W_g1, b_g1, W_g2, b_g2, W_out, b_out):
    E = edge_index.shape[1]
    cpt = -(-E // (_NT * _CHUNK))
    cpt = -(-cpt // _STG) * _STG
    e_pad = _NT * cpt * _CHUNK
    cpt1 = ((2 * cpt * 3 // 4) // _STG) * _STG
    cpt0 = 2 * cpt - cpt1

    src = edge_index[0].astype(jnp.int32)
    dst = edge_index[1].astype(jnp.int32)
    pad = e_pad - E
    src_p = jnp.concatenate([src, jnp.zeros((pad,), jnp.int32)])
    dst_p = jnp.concatenate([dst, jnp.full((pad,), _N, jnp.int32)])
    dst3 = dst_p.reshape(_NT, cpt, _CHUNK)
    n0 = 16 * cpt0 * _CHUNK
    srcA = src_p[:n0].reshape(16, cpt0, _CHUNK)
    dstA = dst_p[:n0].reshape(16, cpt0, _CHUNK)
    srcB = src_p[n0:].reshape(16, cpt1, _CHUNK)
    dstB = dst_p[n0:].reshape(16, cpt1, _CHUNK)

    x = jnp.pad(node_features, ((0, _M - _N), (0, 0)))
    wz1 = W_z[:_D]
    wh1 = W_h[:_D]
    bz = b_z.reshape(1, _D)
    bh = b_h.reshape(1, _D)
    bg1 = b_g1.reshape(1, _D)
    bg2 = b_g2.reshape(1, _D)
    bout = b_out.reshape(1, _D)

    conv = _make_conv(cpt0, cpt1)
    deg = _make_deg(cpt)

    degp = deg(dst3)
    h_gru, g1 = _tc1(x, wz1, bz, wh1, bh, W_g1)
    parts1 = conv(g1, srcA, dstA, srcB, dstB)
    g2 = _tc2(parts1, degp, bg1, W_g2)
    parts2 = conv(g2, srcA, dstA, srcB, dstB)
    out_pad = _tc3(parts2, degp, bg2, h_gru, W_out, bout)
    return out_pad[:_N]


# speedup vs baseline: 1.1728x; 1.1728x over previous
import jax
import jax.numpy as jnp
from jax import lax
from jax.experimental import pallas as pl
from jax.experimental.pallas import tpu as pltpu
from jax.experimental.pallas import tpu_sc as plsc

_N = 10000
_M = 10240
_D = 128
_NT = 32
_CHUNK = 64
_STG = 8
_RPT = _M // 16


def _tc1_body(x_ref, wz_ref, bz_ref, wh_ref, bh_ref, wg1_ref, hgru_ref, g1_ref):
    x = x_ref[...]
    z = jax.nn.sigmoid(
        jnp.dot(x, wz_ref[...], preferred_element_type=jnp.float32) + bz_ref[...])
    ht = jnp.tanh(
        jnp.dot(x, wh_ref[...], preferred_element_type=jnp.float32) + bh_ref[...])
    hg = z * ht
    hgru_ref[...] = hg
    g1_ref[...] = jnp.dot(hg, wg1_ref[...], preferred_element_type=jnp.float32)


def _inv_deg(degp_ref):
    deg = degp_ref[0, :, 0:1] + degp_ref[1, :, 0:1]
    return 1.0 / jnp.maximum(deg, 1.0)


def _tc2_body(p_ref, degp_ref, bg1_ref, wg2_ref, g2_ref):
    inv = _inv_deg(degp_ref)
    h1 = jax.nn.relu((p_ref[0, :, :] + p_ref[1, :, :]) * inv + bg1_ref[...])
    g2_ref[...] = jnp.dot(h1, wg2_ref[...], preferred_element_type=jnp.float32)


def _tc3_body(p_ref, degp_ref, bg2_ref, hgru_ref, wout_ref, bout_ref, out_ref):
    inv = _inv_deg(degp_ref)
    h2 = jax.nn.relu((p_ref[0, :, :] + p_ref[1, :, :]) * inv + bg2_ref[...])
    fused = 0.5 * (hgru_ref[...] + h2)
    out_ref[...] = (
        jnp.dot(fused, wout_ref[...], preferred_element_type=jnp.float32)
        + bout_ref[...])


def _tc1(x, wz, bz, wh, bh, wg1):
    return pl.pallas_call(
        _tc1_body,
        out_shape=(jax.ShapeDtypeStruct((_M, _D), jnp.float32),
                   jax.ShapeDtypeStruct((_M, _D), jnp.float32)),
    )(x, wz, bz, wh, bh, wg1)


def _tc2(parts, degp, bg1, wg2):
    return pl.pallas_call(
        _tc2_body,
        out_shape=jax.ShapeDtypeStruct((_M, _D), jnp.float32),
    )(parts, degp, bg1, wg2)


def _tc3(parts, degp, bg2, hgru, wout, bout):
    return pl.pallas_call(
        _tc3_body,
        out_shape=jax.ShapeDtypeStruct((_M, _D), jnp.float32),
    )(parts, degp, bg2, hgru, wout, bout)


def _sc_mesh():
    return plsc.VectorSubcoreMesh(core_axis_name="c", subcore_axis_name="s")


def _zero_fill(buf, rows):
    zv = jnp.zeros((16,), jnp.float32)

    def zrow(i, carry):
        for j in range(_D // 16):
            buf[i, pl.ds(j * 16, 16)] = zv
        return carry
    lax.fori_loop(0, rows, zrow, 0)


def _make_conv(cpt0: int, cpt1: int):
    assert cpt0 % _STG == 0 and cpt1 % _STG == 0

    scratch = [
        pltpu.VMEM((_STG, _CHUNK), jnp.int32),
        pltpu.VMEM((_STG, _CHUNK), jnp.int32),
        pltpu.VMEM((_CHUNK, _D), jnp.float32),
        pltpu.VMEM((_CHUNK, _D), jnp.float32),
        pltpu.VMEM((_CHUNK, _D), jnp.float32),
        pltpu.VMEM_SHARED((_M, _D), jnp.float32),
        pltpu.SemaphoreType.DMA,
        pltpu.SemaphoreType.DMA,
        pltpu.SemaphoreType.DMA,
        pltpu.SemaphoreType.DMA,
        pltpu.SemaphoreType.DMA,
        pltpu.SemaphoreType.DMA,
    ]

    def body(g_hbm, srcA, dstA, srcB, dstB, part_out, idx_s, idx_d,
             gbuf0, gbuf1, gbuf2, acc,
             gsem0, gsem1, gsem2, ssem0, ssem1, ssem2):
        cid = lax.axis_index("c")
        sid = lax.axis_index("s")
        bufs = (gbuf0, gbuf1, gbuf2)
        gsems = (gsem0, gsem1, gsem2)
        ssems = (ssem0, ssem1, ssem2)

        _zero_fill(gbuf0, _CHUNK)
        base = pl.multiple_of(sid * _RPT, _CHUNK)
        for k in range(_RPT // _CHUNK):
            pltpu.sync_copy(gbuf0, acc.at[pl.ds(base + k * _CHUNK, _CHUNK)])
        plsc.subcore_barrier()

        def run_edges(src_hbm, dst_hbm, nstg):
            def stage_body(s, carry):
                stg = pl.ds(s * _STG, _STG)
                pltpu.sync_copy(src_hbm.at[sid, stg], idx_s)
                pltpu.sync_copy(dst_hbm.at[sid, stg], idx_d)

                g = [None] * _STG
                sc = [None] * _STG
                g[0] = pltpu.async_copy(g_hbm.at[idx_s.at[0]], bufs[0], gsems[0])
                g[1] = pltpu.async_copy(g_hbm.at[idx_s.at[1]], bufs[1], gsems[1])
                for j in range(_STG):
                    p = j % 3
                    g[j].wait()
                    sc[j] = pltpu.async_copy(
                        bufs[p], acc.at[idx_d.at[j]], ssems[p], add=True)
                    if j + 2 < _STG:
                        q = (j + 2) % 3
                        if j >= 1:
                            sc[j - 1].wait()
                        g[j + 2] = pltpu.async_copy(
                            g_hbm.at[idx_s.at[j + 2]], bufs[q], gsems[q])
                sc[_STG - 2].wait()
                sc[_STG - 1].wait()
                return carry
            lax.fori_loop(0, nstg, stage_body, 0)

        @pl.when(cid == 0)
        def _():
            run_edges(srcA, dstA, cpt0 // _STG)

        @pl.when(cid == 1)
        def _():
            run_edges(srcB, dstB, cpt1 // _STG)

        plsc.subcore_barrier()

        for k in range(_RPT // _CHUNK):
            sl = pl.ds(base + k * _CHUNK, _CHUNK)
            pltpu.sync_copy(acc.at[sl], part_out.at[cid, sl])

    return pl.kernel(
        body,
        mesh=_sc_mesh(),
        out_type=jax.ShapeDtypeStruct((2, _M, _D), jnp.float32),
        scratch_types=scratch,
    )


def _make_deg(cpt: int):
    assert cpt % _STG == 0
    nstg = cpt // _STG

    scratch = [
        pltpu.VMEM((_STG, _CHUNK), jnp.int32),
        pltpu.VMEM((_CHUNK, _D), jnp.float32),
        pltpu.VMEM_SHARED((_M, _D), jnp.float32),
        pltpu.SemaphoreType.DMA,
        pltpu.SemaphoreType.DMA,
    ]

    def body(dst_hbm, deg_out, idx_d, obuf, dacc, ssem0, ssem1):
        cid = lax.axis_index("c")
        sid = lax.axis_index("s")
        wid = sid * 2 + cid
        ssems = (ssem0, ssem1)

        _zero_fill(obuf, _CHUNK)
        base = pl.multiple_of(sid * _RPT, _CHUNK)
        for k in range(_RPT // _CHUNK):
            pltpu.sync_copy(obuf, dacc.at[pl.ds(base + k * _CHUNK, _CHUNK)])

        ov = jnp.ones((16,), jnp.float32)

        def orow(i, carry):
            for j in range(_D // 16):
                obuf[i, pl.ds(j * 16, 16)] = ov
            return carry
        lax.fori_loop(0, _CHUNK, orow, 0)
        plsc.subcore_barrier()

        def stage_body(s, carry):
            pltpu.sync_copy(dst_hbm.at[wid, pl.ds(s * _STG, _STG)], idx_d)
            prev = None
            for j in range(_STG):
                cur = pltpu.async_copy(
                    obuf, dacc.at[idx_d.at[j]], ssems[j % 2], add=True)
                if prev is not None:
                    prev.wait()
                prev = cur
            prev.wait()
            return carry
        lax.fori_loop(0, nstg, stage_body, 0)
        plsc.subcore_barrier()

        for k in range(_RPT // _CHUNK):
            sl = pl.ds(base + k * _CHUNK, _CHUNK)
            pltpu.sync_copy(dacc.at[sl], deg_out.at[cid, sl])

    return pl.kernel(
        body,
        mesh=_sc_mesh(),
        out_type=jax.ShapeDtypeStruct((2, _M, _D), jnp.float32),
        scratch_types=scratch,
    )


def kernel(node_features, edge_index, W_r, b_r, W_z, b_z, W_h, b_h,
           W_g1, b_g1, W_g2, b_g2, W_out, b_out):
    E = edge_index.shape[1]
    cpt = -(-E // (_NT * _CHUNK))
    cpt = -(-cpt // _STG) * _STG
    e_pad = _NT * cpt * _CHUNK
    cpt0 = cpt
    cpt1 = cpt

    src = edge_index[0].astype(jnp.int32)
    dst = edge_index[1].astype(jnp.int32)
    pad = e_pad - E
    src_p = jnp.concatenate([src, jnp.zeros((pad,), jnp.int32)])
    dst_p = jnp.concatenate([dst, jnp.full((pad,), _N, jnp.int32)])
    dst3 = dst_p.reshape(_NT, cpt, _CHUNK)
    n0 = 16 * cpt0 * _CHUNK
    srcA = src_p[:n0].reshape(16, cpt0, _CHUNK)
    dstA = dst_p[:n0].reshape(16, cpt0, _CHUNK)
    srcB = src_p[n0:].reshape(16, cpt1, _CHUNK)
    dstB = dst_p[n0:].reshape(16, cpt1, _CHUNK)

    x = jnp.pad(node_features, ((0, _M - _N), (0, 0)))
    wz1 = W_z[:_D]
    wh1 = W_h[:_D]
    bz = b_z.reshape(1, _D)
    bh = b_h.reshape(1, _D)
    bg1 = b_g1.reshape(1, _D)
    bg2 = b_g2.reshape(1, _D)
    bout = b_out.reshape(1, _D)

    conv = _make_conv(cpt0, cpt1)
    deg = _make_deg(cpt)

    degp = deg(dst3)
    h_gru, g1 = _tc1(x, wz1, bz, wh1, bh, W_g1)
    parts1 = conv(g1, srcA, dstA, srcB, dstB)
    g2 = _tc2(parts1, degp, bg1, W_g2)
    parts2 = conv(g2, srcA, dstA, srcB, dstB)
    out_pad = _tc3(parts2, degp, bg2, h_gru, W_out, bout)
    return out_pad[:_N]
